# baseline (speedup 1.0000x reference)
import functools

import jax
import jax.numpy as jnp
from jax import lax
from jax.experimental import pallas as pl
from jax.experimental.pallas import tpu as pltpu
from jax.experimental.pallas import tpu_sc as plsc

_SEQ = 128
_CLS = 101
_SEP = 102
_N_WORDS = 16384
_N_TOK = 32768
_N_LABELS = _N_TOK + 2

_CHUNK = 1024
_WIN = _CHUNK + 16
_NW = 32
_SLOT = 1024
_NGUESS = 3


def _body(st_hbm, seg_hbm, wl_hbm, ids_hbm, mask_hbm, type_hbm, lab_hbm,
          table_v, win_v, out_v, sbuf_v, obuf_v,
          tsem, osem, fsem, ssem):
    c = lax.axis_index("c")
    s = lax.axis_index("s")
    wid = s * 2 + c
    base = wid * _CHUNK

    iota = lax.iota(jnp.int32, 16)
    zero = jnp.zeros((16,), jnp.int32)

    win0 = pl.multiple_of(jnp.maximum(base - 16, 0), 16)
    wcp = pltpu.async_copy(seg_hbm.at[pl.ds(win0, _WIN)], win_v, fsem)

    guess = jnp.clip((base // 2 - _SLOT // 2) // _SLOT, 0,
                     _N_WORDS // _SLOT - _NGUESS)
    for k in range(_NGUESS):
        gsrc = pl.multiple_of((guess + k) * _SLOT, _SLOT)
        pltpu.async_copy(wl_hbm.at[pl.ds(gsrc, _SLOT)],
                         table_v.at[pl.ds(k * _SLOT, _SLOT)], tsem)

    @pl.when(wid == 1)
    def _ids_fetch():
        pltpu.async_copy(st_hbm.at[pl.ds(0, _SEQ)], sbuf_v, ssem)

    @pl.when(wid == 2)
    def _mask():
        one = jnp.ones((16,), jnp.int32)

        def mgrp(i, carry):
            obuf_v[pl.ds(i * 16, 16)] = one
            return carry

        lax.fori_loop(0, _SEQ // 16, mgrp, 0)
        pltpu.async_copy(obuf_v, mask_hbm, ssem)

    @pl.when(wid == 3)
    def _type():
        def zgrp(i, carry):
            obuf_v[pl.ds(i * 16, 16)] = zero
            return carry

        lax.fori_loop(0, _SEQ // 16, zgrp, 0)
        pltpu.async_copy(obuf_v, type_hbm, ssem)

    wcp.wait()

    lo = jnp.min(win_v[pl.ds(0, 16)])
    hi = jnp.max(win_v[pl.ds(_WIN - 16, 16)])
    slot0 = lo // _SLOT
    nslots = hi // _SLOT - slot0 + 1
    ok = jnp.logical_and(slot0 >= guess, hi // _SLOT < guess + _NGUESS)
    slot_base = jnp.where(ok, guess, slot0) * _SLOT

    for k in range(_NGUESS):
        gsrc = pl.multiple_of((guess + k) * _SLOT, _SLOT)
        pltpu.make_async_copy(wl_hbm.at[pl.ds(gsrc, _SLOT)],
                              table_v.at[pl.ds(k * _SLOT, _SLOT)], tsem).wait()

    @pl.when(jnp.logical_not(ok))
    def _refetch():
        def fire_slot(k, carry):
            src = pl.multiple_of((slot0 + k) * _SLOT, _SLOT)
            pltpu.async_copy(wl_hbm.at[pl.ds(src, _SLOT)],
                             table_v.at[pl.ds(k * _SLOT, _SLOT)], tsem)
            return carry

        lax.fori_loop(0, nslots, fire_slot, 0)

        def drain_slot(k, carry):
            src = pl.multiple_of((slot0 + k) * _SLOT, _SLOT)
            pltpu.make_async_copy(wl_hbm.at[pl.ds(src, _SLOT)],
                                  table_v.at[pl.ds(k * _SLOT, _SLOT)],
                                  tsem).wait()
            return carry

        lax.fori_loop(0, nslots, drain_slot, 0)

    half = _CHUNK // 2

    @plsc.parallel_loop(0, 32, unroll=8)
    def gather_a(i):
        p = base + i * 16 + iota
        loc = jnp.clip(p - 1 - win0, 0, _WIN - 1)
        segv = plsc.load_gather(win_v, [loc])
        vals = plsc.load_gather(table_v, [segv - slot_base]) + 1
        out_v[pl.ds(i * 16, 16)] = jnp.where(p == 0, zero, vals)

    cp_a = pltpu.async_copy(out_v.at[pl.ds(0, half)],
                            lab_hbm.at[pl.ds(base, half)], osem)

    @plsc.parallel_loop(32, _WIN // 16, unroll=11)
    def gather_b(i):
        p = base + i * 16 + iota
        loc = jnp.clip(p - 1 - win0, 0, _WIN - 1)
        segv = plsc.load_gather(win_v, [loc])
        vals = plsc.load_gather(table_v, [segv - slot_base]) + 1
        out_v[pl.ds(i * 16, 16)] = jnp.where(p >= _N_LABELS - 1, zero, vals)

    cp_b = pltpu.async_copy(out_v.at[pl.ds(half, half)],
                            lab_hbm.at[pl.ds(base + half, half)], osem)

    @pl.when(wid == 1)
    def _ids():
        pltpu.make_async_copy(st_hbm.at[pl.ds(0, _SEQ)], sbuf_v, ssem).wait()

        def idgrp(i, carry):
            p = i * 16 + iota
            loc = jnp.clip(p - 1, 0, _SEQ - 1)
            v = plsc.load_gather(sbuf_v, [loc])
            v = jnp.where(p == 0, jnp.full((16,), _CLS, jnp.int32), v)
            v = jnp.where(p == _SEQ - 1, jnp.full((16,), _SEP, jnp.int32), v)
            obuf_v[pl.ds(i * 16, 16)] = v
            return carry

        lax.fori_loop(0, _SEQ // 16, idgrp, 0)
        pltpu.async_copy(obuf_v, ids_hbm, ssem)

    @pl.when(wid == _NW - 1)
    def _tail():
        pltpu.async_copy(out_v.at[pl.ds(_CHUNK, 2)],
                         lab_hbm.at[pl.ds(_NW * _CHUNK, 2)], osem)

    cp_a.wait()
    cp_b.wait()

    @pl.when(wid == _NW - 1)
    def _tail_wait():
        pltpu.make_async_copy(out_v.at[pl.ds(_CHUNK, 2)],
                              lab_hbm.at[pl.ds(_NW * _CHUNK, 2)], osem).wait()

    @pl.when(wid == 1)
    def _ids_wait():
        pltpu.make_async_copy(obuf_v, ids_hbm, ssem).wait()

    @pl.when(wid == 2)
    def _mask_wait():
        pltpu.make_async_copy(obuf_v, mask_hbm, ssem).wait()

    @pl.when(wid == 3)
    def _type_wait():
        pltpu.make_async_copy(obuf_v, type_hbm, ssem).wait()


@jax.jit
def _run(subtoken_ids, seg32, wl32):
    i32 = jnp.int32
    k = functools.partial(
        pl.kernel,
        out_type=(
            jax.ShapeDtypeStruct((_SEQ,), i32),
            jax.ShapeDtypeStruct((_SEQ,), i32),
            jax.ShapeDtypeStruct((_SEQ,), i32),
            jax.ShapeDtypeStruct((_N_LABELS,), i32),
        ),
        mesh=plsc.VectorSubcoreMesh(core_axis_name="c", subcore_axis_name="s"),
        compiler_params=pltpu.CompilerParams(needs_layout_passes=False,
                                             skip_device_barrier=True),
        scratch_types=[
            pltpu.VMEM((_N_WORDS,), i32),
            pltpu.VMEM((_WIN,), i32),
            pltpu.VMEM((_WIN,), i32),
            pltpu.VMEM((_SEQ,), i32),
            pltpu.VMEM((_SEQ,), i32),
            pltpu.SemaphoreType.DMA,
            pltpu.SemaphoreType.DMA,
            pltpu.SemaphoreType.DMA,
            pltpu.SemaphoreType.DMA,
        ],
    )(_body)
    return k(subtoken_ids, seg32, wl32)


def kernel(subtoken_ids, segment_ids, word_labels):
    seg32 = segment_ids.astype(jnp.int32)
    wl32 = word_labels.astype(jnp.int32)
    return _run(subtoken_ids, seg32, wl32)


# speedup vs baseline: 1.0088x; 1.0088x over previous
import functools

import jax
import jax.numpy as jnp
from jax import lax
from jax.experimental import pallas as pl
from jax.experimental.pallas import tpu as pltpu
from jax.experimental.pallas import tpu_sc as plsc

_SEQ = 128
_CLS = 101
_SEP = 102
_N_WORDS = 16384
_N_TOK = 32768
_N_LABELS = _N_TOK + 2

_CHUNK = 1024
_WIN = _CHUNK + 16
_NW = 32
_SLOT = 1024
_NGUESS = 3


def _body(st_hbm, seg_hbm, wl_hbm, ids_hbm, mask_hbm, type_hbm, lab_hbm,
          table_v, win_v, idx_v, out_v, sbuf_v, obuf_v,
          tsem, osem, fsem, ssem):
    c = lax.axis_index("c")
    s = lax.axis_index("s")
    wid = s * 2 + c
    base = wid * _CHUNK

    iota = lax.iota(jnp.int32, 16)
    zero = jnp.zeros((16,), jnp.int32)

    win0 = pl.multiple_of(jnp.maximum(base - 16, 0), 16)
    wcp = pltpu.async_copy(seg_hbm.at[pl.ds(win0, _WIN)], win_v, fsem)

    guess = jnp.clip((base // 2 - _SLOT // 2) // _SLOT, 0,
                     _N_WORDS // _SLOT - _NGUESS)
    for k in range(_NGUESS):
        gsrc = pl.multiple_of((guess + k) * _SLOT, _SLOT)
        pltpu.async_copy(wl_hbm.at[pl.ds(gsrc, _SLOT)],
                         table_v.at[pl.ds(k * _SLOT, _SLOT)], tsem)

    @pl.when(wid == 1)
    def _ids_fetch():
        pltpu.async_copy(st_hbm.at[pl.ds(0, _SEQ)], sbuf_v, ssem)

    @pl.when(wid == 2)
    def _mask():
        one = jnp.ones((16,), jnp.int32)

        def mgrp(i, carry):
            obuf_v[pl.ds(i * 16, 16)] = one
            return carry

        lax.fori_loop(0, _SEQ // 16, mgrp, 0)
        pltpu.async_copy(obuf_v, mask_hbm, ssem)

    @pl.when(wid == 3)
    def _type():
        def zgrp(i, carry):
            obuf_v[pl.ds(i * 16, 16)] = zero
            return carry

        lax.fori_loop(0, _SEQ // 16, zgrp, 0)
        pltpu.async_copy(obuf_v, type_hbm, ssem)

    wcp.wait()

    lo = jnp.min(win_v[pl.ds(0, 16)])
    hi = jnp.max(win_v[pl.ds(_WIN - 16, 16)])
    slot0 = lo // _SLOT
    nslots = hi // _SLOT - slot0 + 1
    ok = jnp.logical_and(slot0 >= guess, hi // _SLOT < guess + _NGUESS)
    slot_base = jnp.where(ok, guess, slot0) * _SLOT

    @plsc.parallel_loop(0, _WIN // 16, unroll=5)
    def build(i):
        p = base + i * 16 + iota
        loc = jnp.clip(p - 1 - win0, 0, _WIN - 1)
        idx_v[pl.ds(i * 16, 16)] = plsc.load_gather(win_v, [loc])

    for k in range(_NGUESS):
        gsrc = pl.multiple_of((guess + k) * _SLOT, _SLOT)
        pltpu.make_async_copy(wl_hbm.at[pl.ds(gsrc, _SLOT)],
                              table_v.at[pl.ds(k * _SLOT, _SLOT)], tsem).wait()

    @pl.when(jnp.logical_not(ok))
    def _refetch():
        def fire_slot(k, carry):
            src = pl.multiple_of((slot0 + k) * _SLOT, _SLOT)
            pltpu.async_copy(wl_hbm.at[pl.ds(src, _SLOT)],
                             table_v.at[pl.ds(k * _SLOT, _SLOT)], tsem)
            return carry

        lax.fori_loop(0, nslots, fire_slot, 0)

        def drain_slot(k, carry):
            src = pl.multiple_of((slot0 + k) * _SLOT, _SLOT)
            pltpu.make_async_copy(wl_hbm.at[pl.ds(src, _SLOT)],
                                  table_v.at[pl.ds(k * _SLOT, _SLOT)],
                                  tsem).wait()
            return carry

        lax.fori_loop(0, nslots, drain_slot, 0)

    half = _CHUNK // 2

    @plsc.parallel_loop(0, 32, unroll=8)
    def gather_a(i):
        p = base + i * 16 + iota
        segv = idx_v[pl.ds(i * 16, 16)]
        vals = plsc.load_gather(table_v, [segv - slot_base]) + 1
        out_v[pl.ds(i * 16, 16)] = jnp.where(p == 0, zero, vals)

    cp_a = pltpu.async_copy(out_v.at[pl.ds(0, half)],
                            lab_hbm.at[pl.ds(base, half)], osem)

    @plsc.parallel_loop(32, _WIN // 16, unroll=11)
    def gather_b(i):
        p = base + i * 16 + iota
        segv = idx_v[pl.ds(i * 16, 16)]
        vals = plsc.load_gather(table_v, [segv - slot_base]) + 1
        out_v[pl.ds(i * 16, 16)] = jnp.where(p >= _N_LABELS - 1, zero, vals)

    cp_b = pltpu.async_copy(out_v.at[pl.ds(half, half)],
                            lab_hbm.at[pl.ds(base + half, half)], osem)

    @pl.when(wid == 1)
    def _ids():
        pltpu.make_async_copy(st_hbm.at[pl.ds(0, _SEQ)], sbuf_v, ssem).wait()

        def idgrp(i, carry):
            p = i * 16 + iota
            loc = jnp.clip(p - 1, 0, _SEQ - 1)
            v = plsc.load_gather(sbuf_v, [loc])
            v = jnp.where(p == 0, jnp.full((16,), _CLS, jnp.int32), v)
            v = jnp.where(p == _SEQ - 1, jnp.full((16,), _SEP, jnp.int32), v)
            obuf_v[pl.ds(i * 16, 16)] = v
            return carry

        lax.fori_loop(0, _SEQ // 16, idgrp, 0)
        pltpu.async_copy(obuf_v, ids_hbm, ssem)

    @pl.when(wid == _NW - 1)
    def _tail():
        pltpu.async_copy(out_v.at[pl.ds(_CHUNK, 2)],
                         lab_hbm.at[pl.ds(_NW * _CHUNK, 2)], osem)

    cp_a.wait()
    cp_b.wait()

    @pl.when(wid == _NW - 1)
    def _tail_wait():
        pltpu.make_async_copy(out_v.at[pl.ds(_CHUNK, 2)],
                              lab_hbm.at[pl.ds(_NW * _CHUNK, 2)], osem).wait()

    @pl.when(wid == 1)
    def _ids_wait():
        pltpu.make_async_copy(obuf_v, ids_hbm, ssem).wait()

    @pl.when(wid == 2)
    def _mask_wait():
        pltpu.make_async_copy(obuf_v, mask_hbm, ssem).wait()

    @pl.when(wid == 3)
    def _type_wait():
        pltpu.make_async_copy(obuf_v, type_hbm, ssem).wait()


@jax.jit
def _run(subtoken_ids, seg32, wl32):
    i32 = jnp.int32
    k = functools.partial(
        pl.kernel,
        out_type=(
            jax.ShapeDtypeStruct((_SEQ,), i32),
            jax.ShapeDtypeStruct((_SEQ,), i32),
            jax.ShapeDtypeStruct((_SEQ,), i32),
            jax.ShapeDtypeStruct((_N_LABELS,), i32),
        ),
        mesh=plsc.VectorSubcoreMesh(core_axis_name="c", subcore_axis_name="s"),
        compiler_params=pltpu.CompilerParams(needs_layout_passes=False,
                                             skip_device_barrier=True),
        scratch_types=[
            pltpu.VMEM((_N_WORDS,), i32),
            pltpu.VMEM((_WIN,), i32),
            pltpu.VMEM((_WIN,), i32),
            pltpu.VMEM((_WIN,), i32),
            pltpu.VMEM((_SEQ,), i32),
            pltpu.VMEM((_SEQ,), i32),
            pltpu.SemaphoreType.DMA,
            pltpu.SemaphoreType.DMA,
            pltpu.SemaphoreType.DMA,
            pltpu.SemaphoreType.DMA,
        ],
    )(_body)
    return k(subtoken_ids, seg32, wl32)


def kernel(subtoken_ids, segment_ids, word_labels):
    seg32 = segment_ids.astype(jnp.int32)
    wl32 = word_labels.astype(jnp.int32)
    return _run(subtoken_ids, seg32, wl32)
